# Initial kernel scaffold; baseline (speedup 1.0000x reference)
import jax
import jax.numpy as jnp
from jax.experimental import pallas as pl


def kernel(x, conv_weight):
    raise NotImplementedError("write your pallas kernel here")


# speedup vs baseline: 1.1853x; 1.1853x over previous
import functools

import jax
import jax.numpy as jnp
from jax.experimental import pallas as pl
from jax.experimental.pallas import tpu as pltpu


def _eca_fused_kernel(w_ref, x_ref, o_ref, *, k, pad, inv_hw):
    x = x_ref[...]
    xf = x.astype(jnp.float32)
    mean = jnp.sum(xf, axis=2, keepdims=True) * inv_hw
    nb = mean.shape[0]
    conv = jnp.zeros_like(mean)
    for t in range(k):
        d = t - pad
        if d == 0:
            sh = mean
        elif d > 0:
            sh = jnp.concatenate(
                [mean[:, d:, :], jnp.zeros((nb, d, 1), mean.dtype)], axis=1)
        else:
            sh = jnp.concatenate(
                [jnp.zeros((nb, -d, 1), mean.dtype), mean[:, :d, :]], axis=1)
        conv = conv + w_ref[t] * sh
    gate = jax.nn.sigmoid(conv)
    o_ref[...] = (xf * gate).astype(o_ref.dtype)


def kernel(x, conv_weight):
    B, C, H, W = x.shape
    HW = H * W
    k = conv_weight.shape[0]
    pad = (k - 1) // 2
    w32 = jnp.asarray(conv_weight, jnp.float32).reshape(-1)

    nb = 4
    while B % nb != 0:
        nb //= 2
    x3 = x.reshape(B, C, HW)

    out3 = pl.pallas_call(
        functools.partial(_eca_fused_kernel, k=k, pad=pad, inv_hw=1.0 / HW),
        out_shape=jax.ShapeDtypeStruct((B, C, HW), x.dtype),
        grid_spec=pltpu.PrefetchScalarGridSpec(
            num_scalar_prefetch=1,
            grid=(B // nb,),
            in_specs=[pl.BlockSpec((nb, C, HW), lambda b, w: (b, 0, 0))],
            out_specs=pl.BlockSpec((nb, C, HW), lambda b, w: (b, 0, 0))),
        compiler_params=pltpu.CompilerParams(
            dimension_semantics=("parallel",),
            vmem_limit_bytes=48 * 1024 * 1024),
    )(w32, x3)
    return out3.reshape(B, C, H, W)


# speedup vs baseline: 1.1864x; 1.0009x over previous
import functools

import jax
import jax.numpy as jnp
from jax.experimental import pallas as pl
from jax.experimental.pallas import tpu as pltpu


def _eca_fused_kernel(w_ref, x_ref, o_ref, *, k, pad, inv_hw):
    x = x_ref[...]
    xf = x.astype(jnp.float32)
    mean = jnp.sum(xf, axis=2, keepdims=True) * inv_hw
    nb = mean.shape[0]
    conv = jnp.zeros_like(mean)
    for t in range(k):
        d = t - pad
        if d == 0:
            sh = mean
        elif d > 0:
            sh = jnp.concatenate(
                [mean[:, d:, :], jnp.zeros((nb, d, 1), mean.dtype)], axis=1)
        else:
            sh = jnp.concatenate(
                [jnp.zeros((nb, -d, 1), mean.dtype), mean[:, :d, :]], axis=1)
        conv = conv + w_ref[t] * sh
    gate = jax.nn.sigmoid(conv)
    o_ref[...] = (xf * gate).astype(o_ref.dtype)


def kernel(x, conv_weight):
    B, C, H, W = x.shape
    HW = H * W
    k = conv_weight.shape[0]
    pad = (k - 1) // 2
    w32 = jnp.asarray(conv_weight, jnp.float32).reshape(-1)

    nb = 8
    while B % nb != 0:
        nb //= 2
    x3 = x.reshape(B, C, HW)

    out3 = pl.pallas_call(
        functools.partial(_eca_fused_kernel, k=k, pad=pad, inv_hw=1.0 / HW),
        out_shape=jax.ShapeDtypeStruct((B, C, HW), x.dtype),
        grid_spec=pltpu.PrefetchScalarGridSpec(
            num_scalar_prefetch=1,
            grid=(B // nb,),
            in_specs=[pl.BlockSpec((nb, C, HW), lambda b, w: (b, 0, 0))],
            out_specs=pl.BlockSpec((nb, C, HW), lambda b, w: (b, 0, 0))),
        compiler_params=pltpu.CompilerParams(
            dimension_semantics=("parallel",),
            vmem_limit_bytes=48 * 1024 * 1024),
    )(w32, x3)
    return out3.reshape(B, C, H, W)
